# baseline (speedup 1.0000x reference)
import jax
import jax.numpy as jnp
from jax import lax
from jax.experimental import pallas as pl
from jax.experimental.pallas import tpu as pltpu
from jax.experimental.pallas import tpu_sc as plsc

N = 10000
E = 160000
D_IN = 256
HEADS = 4
C_OUT = 256
NEG_SLOPE = 0.2

NC, NS, LANES = 2, 16, 16
NPAD = 10240
SLT = NPAD // NS
EPAD = E + NPAD
ERT = E // NS
RCHUNK = 2000
FB = 8
BROWS = 80


def _iota16():
    return lax.iota(jnp.int32, LANES)


def _matmul_body(x_ref, w_ref, o_ref, a_ref):
    o_ref[0] = jnp.dot(x_ref[...], w_ref[...],
                       preferred_element_type=jnp.float32)
    @pl.when(pl.program_id(1) == FB)
    def _():
        a_ref[0] = jnp.dot(x_ref[...], w_ref[:, 0:4],
                           preferred_element_type=jnp.float32)
        a_ref[1] = jnp.dot(x_ref[...], w_ref[:, 4:8],
                           preferred_element_type=jnp.float32)


def _edge_alpha(a_v, sidx, didx, k):
    a_s = plsc.load_gather(a_v, [sidx * 4 + k])
    a_d = plsc.load_gather(a_v, [didx * 4 + (2 + k)])
    al = a_s + a_d
    al = jnp.maximum(al, NEG_SLOPE * al)
    return jnp.exp(al)


def _softmax_body(a_hbm, src_hbm, dst_hbm, w_hbm,
                  a_v, dn_v, src_v, dst_v, wbuf_v, tmp_v, red_v, den_sh):
    c = lax.axis_index("c")
    s = lax.axis_index("s")
    DN = NPAD * 2
    SLICE = DN // NS

    pltpu.sync_copy(a_hbm.at[pl.ds(c * (N * 4), N * 4)], a_v)

    def _zero_dn(i, _):
        dn_v[pl.ds(i * LANES, LANES)] = jnp.zeros((LANES,), jnp.float32)
        return 0
    lax.fori_loop(0, DN // LANES, _zero_dn, 0)

    def _chunk1(ci, _):
        e0 = s * ERT + ci * RCHUNK
        pltpu.sync_copy(src_hbm.at[pl.ds(e0, RCHUNK)], src_v)
        pltpu.sync_copy(dst_hbm.at[pl.ds(e0, RCHUNK)], dst_v)

        def _vec(vi, _):
            sidx = src_v[pl.ds(vi * LANES, LANES)]
            didx = dst_v[pl.ds(vi * LANES, LANES)]
            for k in range(2):
                ea = _edge_alpha(a_v, sidx, didx, k)
                plsc.addupdate_scatter(dn_v, [didx * 2 + k], ea)
            return 0
        lax.fori_loop(0, RCHUNK // LANES, _vec, 0)
        return 0
    lax.fori_loop(0, ERT // RCHUNK, _chunk1, 0)

    def _slvec(vi, _):
        q = s * SLT + vi * LANES + _iota16()
        valid = q < N
        qc = jnp.where(valid, q, N - 1)
        for k in range(2):
            ea = _edge_alpha(a_v, qc, qc, k)
            ea = jnp.where(valid, ea, 0.0)
            plsc.addupdate_scatter(dn_v, [qc * 2 + k], ea)
        return 0
    lax.fori_loop(0, SLT // LANES, _slvec, 0)

    pltpu.sync_copy(dn_v, den_sh.at[pl.ds(s * DN, DN)])
    plsc.subcore_barrier()
    off = s * SLICE
    pltpu.sync_copy(den_sh.at[pl.ds(off, SLICE)], red_v)

    def _racc(t, _):
        pltpu.sync_copy(den_sh.at[pl.ds(t * DN + off, SLICE)], tmp_v)
        def _vadd(v, _):
            sl = pl.ds(v * LANES, LANES)
            red_v[sl] = red_v[sl] + tmp_v[sl]
            return 0
        lax.fori_loop(0, SLICE // LANES, _vadd, 0)
        return 0
    lax.fori_loop(1, NS, _racc, 0)

    pltpu.sync_copy(red_v, den_sh.at[pl.ds(NS * DN + off, SLICE)])
    plsc.subcore_barrier()
    pltpu.sync_copy(den_sh.at[pl.ds(NS * DN, DN)], dn_v)

    def _chunk2(ci, _):
        e0 = s * ERT + ci * RCHUNK
        pltpu.sync_copy(src_hbm.at[pl.ds(e0, RCHUNK)], src_v)
        pltpu.sync_copy(dst_hbm.at[pl.ds(e0, RCHUNK)], dst_v)

        def _vec(vi, _):
            sidx = src_v[pl.ds(vi * LANES, LANES)]
            didx = dst_v[pl.ds(vi * LANES, LANES)]
            for k in range(2):
                ea = _edge_alpha(a_v, sidx, didx, k)
                den = plsc.load_gather(dn_v, [didx * 2 + k])
                wbuf_v[pl.ds(k * RCHUNK + vi * LANES, LANES)] = ea / den
            return 0
        lax.fori_loop(0, RCHUNK // LANES, _vec, 0)
        for k in range(2):
            pltpu.sync_copy(wbuf_v.at[pl.ds(k * RCHUNK, RCHUNK)],
                            w_hbm.at[pl.ds((2 * c + k) * EPAD + e0, RCHUNK)])
        return 0
    lax.fori_loop(0, ERT // RCHUNK, _chunk2, 0)

    def _slvec2(vi, _):
        q = s * SLT + vi * LANES + _iota16()
        valid = q < N
        qc = jnp.where(valid, q, N - 1)
        for k in range(2):
            ea = _edge_alpha(a_v, qc, qc, k)
            den = plsc.load_gather(dn_v, [qc * 2 + k])
            w = jnp.where(valid, ea / den, 0.0)
            wbuf_v[pl.ds(k * RCHUNK + vi * LANES, LANES)] = w
        return 0
    lax.fori_loop(0, SLT // LANES, _slvec2, 0)
    for k in range(2):
        pltpu.sync_copy(
            wbuf_v.at[pl.ds(k * RCHUNK, SLT)],
            w_hbm.at[pl.ds((2 * c + k) * EPAD + E + s * SLT, SLT)])


def _agg_body(xp_hbm, src_hbm, dst_hbm, w_hbm, bias_hbm, out_hbm,
              w_v, wsl_v, rows_a, rows_b,
              gq0a, gq0b, gq1a, gq1b, sq0a, sq0b, sq1a, sq1b,
              bias_v, sem_i0a, sem_i0b, sem_i1a, sem_i1b,
              sem_d0a, sem_d0b, sem_d1a, sem_d1b,
              sem_ga, sem_gb, sem_sa, sem_sb, acc_sh):
    c = lax.axis_index("c")
    s = lax.axis_index("s")
    r0 = s * SLT
    e_base = s * ERT
    NB = ERT // BROWS
    NGI = (NB - 1) // 4

    def _scale_rows(rows, wref, b0):
        def _scale(rh, _):
            r = rh * 2
            wv0 = plsc.load_gather(
                wref, [jnp.full((LANES,), b0, jnp.int32) + r])
            wv1 = plsc.load_gather(
                wref, [jnp.full((LANES,), b0 + 1, jnp.int32) + r])
            for k in range(8):
                sl = pl.ds(k * LANES, LANES)
                rows[r, sl] = rows[r, sl] * wv0
            for k in range(8):
                sl = pl.ds(k * LANES, LANES)
                rows[r + 1, sl] = rows[r + 1, sl] * wv1
            return 0
        lax.fori_loop(0, BROWS // 2, _scale, 0)

    def _add_fbn(gq, fb):
        def _add(v, _):
            sl = pl.ds(v * LANES, LANES)
            gq[sl] = gq[sl] + fb * N
            return 0
        lax.fori_loop(0, BROWS // LANES, _add, 0)

    def _pass(j, _):
        fb = 4 * c + j
        h = fb // 2

        pltpu.sync_copy(w_hbm.at[pl.ds(h * EPAD + e_base, ERT)], w_v)
        pltpu.sync_copy(w_hbm.at[pl.ds(h * EPAD + E + s * SLT, SLT)], wsl_v)
        pltpu.sync_copy(bias_hbm.at[pl.ds(fb * 128, 128)], bias_v)

        def _sl_init(wbase, count_half):
            def _one(rh, _):
                r = rh * 2
                wv0 = plsc.load_gather(
                    wsl_v, [jnp.full((LANES,), wbase, jnp.int32) + r])
                wv1 = plsc.load_gather(
                    wsl_v, [jnp.full((LANES,), wbase + 1, jnp.int32) + r])
                for k in range(8):
                    sl = pl.ds(k * LANES, LANES)
                    rows_a[r, sl] = rows_a[r, sl] * wv0 + bias_v[sl]
                for k in range(8):
                    sl = pl.ds(k * LANES, LANES)
                    rows_a[r + 1, sl] = rows_a[r + 1, sl] * wv1 + bias_v[sl]
                return 0
            lax.fori_loop(0, count_half, _one, 0)

        def _init(zi, _):
            rr = r0 + zi * 64
            @pl.when(rr + 64 <= N)
            def _():
                pltpu.sync_copy(xp_hbm.at[pl.ds(fb * N + rr, 64)],
                                rows_a.at[pl.ds(0, 64)])
                _sl_init(zi * 64, 32)
                pltpu.sync_copy(rows_a.at[pl.ds(0, 64)],
                                acc_sh.at[pl.ds(rr, 64)])
            @pl.when(rr == (N // 64) * 64)
            def _():
                nrem = N - (N // 64) * 64
                pltpu.sync_copy(xp_hbm.at[pl.ds(fb * N + rr, nrem)],
                                rows_a.at[pl.ds(0, nrem)])
                _sl_init(zi * 64, nrem // 2)
                pltpu.sync_copy(rows_a.at[pl.ds(0, nrem)],
                                acc_sh.at[pl.ds(rr, nrem)])
            return 0
        lax.fori_loop(0, SLT // 64, _init, 0)
        plsc.subcore_barrier()

        def _pre_g(b, gq, si):
            pltpu.async_copy(
                src_hbm.at[pl.ds(e_base + b * BROWS, BROWS)], gq, si)

        def _pre_s(b, sq, sd):
            pltpu.async_copy(
                dst_hbm.at[pl.ds(e_base + b * BROWS, BROWS)], sq, sd)

        _pre_g(0, gq0a, sem_i0a); _pre_s(0, sq0a, sem_d0a)
        _pre_g(1, gq0b, sem_i0b); _pre_s(1, sq0b, sem_d0b)
        _pre_g(2, gq1a, sem_i1a); _pre_s(2, sq1a, sem_d1a)
        _pre_g(3, gq1b, sem_i1b); _pre_s(3, sq1b, sem_d1b)

        def _group(g, _):
            b0 = 4 * g

            @pl.when(g > 0)
            def _():
                pltpu.make_async_copy(rows_a, acc_sh.at[sq1a], sem_sa).wait()
                _pre_s(b0 + 2, sq1a, sem_d1a)
                pltpu.make_async_copy(rows_b, acc_sh.at[sq1b], sem_sb).wait()
                _pre_s(b0 + 3, sq1b, sem_d1b)

            def _wait_g(gq, si, boff):
                e0 = e_base + (b0 + boff) * BROWS
                pltpu.make_async_copy(
                    src_hbm.at[pl.ds(e0, BROWS)], gq, si).wait()

            def _wait_s(sq, sd, boff):
                e0 = e_base + (b0 + boff) * BROWS
                pltpu.make_async_copy(
                    dst_hbm.at[pl.ds(e0, BROWS)], sq, sd).wait()

            more = g + 1 < NGI

            _wait_g(gq0a, sem_i0a, 0)
            _add_fbn(gq0a, fb)
            d_ga = pltpu.async_copy(xp_hbm.at[gq0a], rows_a, sem_ga)
            _wait_g(gq0b, sem_i0b, 1)
            _add_fbn(gq0b, fb)
            d_gb = pltpu.async_copy(xp_hbm.at[gq0b], rows_b, sem_gb)
            d_ga.wait()
            @pl.when(more)
            def _():
                _pre_g(b0 + 4, gq0a, sem_i0a)
            _scale_rows(rows_a, w_v, b0 * BROWS)
            _wait_s(sq0a, sem_d0a, 0)
            d_sa = pltpu.async_copy(rows_a, acc_sh.at[sq0a], sem_sa, add=True)
            d_gb.wait()
            @pl.when(more)
            def _():
                _pre_g(b0 + 5, gq0b, sem_i0b)
            _scale_rows(rows_b, w_v, (b0 + 1) * BROWS)
            _wait_s(sq0b, sem_d0b, 1)
            d_sb = pltpu.async_copy(rows_b, acc_sh.at[sq0b], sem_sb, add=True)

            _wait_g(gq1a, sem_i1a, 2)
            _add_fbn(gq1a, fb)
            d_sa.wait()
            @pl.when(more)
            def _():
                _pre_s(b0 + 4, sq0a, sem_d0a)
            d_gc = pltpu.async_copy(xp_hbm.at[gq1a], rows_a, sem_ga)
            _wait_g(gq1b, sem_i1b, 3)
            _add_fbn(gq1b, fb)
            d_sb.wait()
            @pl.when(more)
            def _():
                _pre_s(b0 + 5, sq0b, sem_d0b)
            d_gd = pltpu.async_copy(xp_hbm.at[gq1b], rows_b, sem_gb)
            d_gc.wait()
            @pl.when(more)
            def _():
                _pre_g(b0 + 6, gq1a, sem_i1a)
            _scale_rows(rows_a, w_v, (b0 + 2) * BROWS)
            _wait_s(sq1a, sem_d1a, 2)
            pltpu.async_copy(rows_a, acc_sh.at[sq1a], sem_sa, add=True)
            d_gd.wait()
            @pl.when(more)
            def _():
                _pre_g(b0 + 7, gq1b, sem_i1b)
            _scale_rows(rows_b, w_v, (b0 + 3) * BROWS)
            _wait_s(sq1b, sem_d1b, 3)
            pltpu.async_copy(rows_b, acc_sh.at[sq1b], sem_sb, add=True)
            return 0
        lax.fori_loop(0, NGI, _group, 0)

        pltpu.make_async_copy(rows_a, acc_sh.at[sq1a], sem_sa).wait()
        pltpu.make_async_copy(rows_b, acc_sh.at[sq1b], sem_sb).wait()

        bT = NB - 1
        eT = e_base + bT * BROWS
        pltpu.sync_copy(src_hbm.at[pl.ds(eT, BROWS)], gq0a)
        pltpu.sync_copy(dst_hbm.at[pl.ds(eT, BROWS)], sq0a)
        _add_fbn(gq0a, fb)
        pltpu.sync_copy(xp_hbm.at[gq0a], rows_a)
        _scale_rows(rows_a, w_v, bT * BROWS)
        pltpu.sync_copy(rows_a, acc_sh.at[sq0a], add=True)

        plsc.subcore_barrier()

        def _wchunk(zi, _):
            rr = r0 + zi * 64
            @pl.when(rr + 64 <= N)
            def _():
                pltpu.sync_copy(acc_sh.at[pl.ds(rr, 64)],
                                out_hbm.at[pl.ds(rr, 64),
                                           pl.ds(fb * 128, 128)])
            @pl.when(rr == (N // 64) * 64)
            def _():
                nrem = N - (N // 64) * 64
                pltpu.sync_copy(acc_sh.at[pl.ds(rr, nrem)],
                                out_hbm.at[pl.ds(rr, nrem),
                                           pl.ds(fb * 128, 128)])
            return 0
        lax.fori_loop(0, SLT // 64, _wchunk, 0)
        plsc.subcore_barrier()
        return 0
    lax.fori_loop(0, 4, _pass, 0)


@jax.jit
def kernel(x, edge_index, W, att_src, att_dst, bias):
    ei = edge_index.astype(jnp.int32)
    src = ei[0]
    dst = ei[1]

    w3 = W.reshape(D_IN, HEADS, C_OUT)
    weff_src = jnp.einsum('dhc,hc->dh', w3, att_src)
    weff_dst = jnp.einsum('dhc,hc->dh', w3, att_dst)
    wsel = jnp.concatenate(
        [weff_src[:, 0:2], weff_dst[:, 0:2],
         weff_src[:, 2:4], weff_dst[:, 2:4]], axis=1)
    wcat = jnp.concatenate(
        [W, wsel, jnp.zeros((D_IN, 120), jnp.float32)], axis=1)

    RT = 1000
    xp_all, a_sc = pl.pallas_call(
        _matmul_body,
        grid=(N // RT, FB + 1),
        in_specs=[
            pl.BlockSpec((RT, D_IN), lambda i, j: (i, 0)),
            pl.BlockSpec((D_IN, 128), lambda i, j: (0, j)),
        ],
        out_specs=[
            pl.BlockSpec((1, RT, 128), lambda i, j: (j, i, 0)),
            pl.BlockSpec((2, RT, 4), lambda i, j: (0, i, 0)),
        ],
        out_shape=[
            jax.ShapeDtypeStruct((FB + 1, N, 128), jnp.float32),
            jax.ShapeDtypeStruct((2, N, 4), jnp.float32),
        ],
    )(x, wcat)

    a_all = a_sc.reshape(2 * N * 4)
    xp_flat = xp_all.reshape((FB + 1) * N, 128)

    mesh = plsc.VectorSubcoreMesh(core_axis_name="c", subcore_axis_name="s",
                                  num_cores=NC, num_subcores=NS)

    w_all = pl.kernel(
        _softmax_body,
        out_type=jax.ShapeDtypeStruct((HEADS * EPAD,), jnp.float32),
        mesh=mesh,
        compiler_params=pltpu.CompilerParams(needs_layout_passes=False),
        scratch_types=[
            pltpu.VMEM((N * 4,), jnp.float32),
            pltpu.VMEM((NPAD * 2,), jnp.float32),
            pltpu.VMEM((RCHUNK,), jnp.int32),
            pltpu.VMEM((RCHUNK,), jnp.int32),
            pltpu.VMEM((2 * RCHUNK,), jnp.float32),
            pltpu.VMEM((NPAD * 2 // NS,), jnp.float32),
            pltpu.VMEM((NPAD * 2 // NS,), jnp.float32),
            pltpu.VMEM_SHARED(((NS + 1) * NPAD * 2,), jnp.float32),
        ],
    )(a_all, src, dst)

    out = pl.kernel(
        _agg_body,
        out_type=jax.ShapeDtypeStruct((N, HEADS * C_OUT), jnp.float32),
        mesh=mesh,
        compiler_params=pltpu.CompilerParams(needs_layout_passes=False),
        scratch_types=[
            pltpu.VMEM((ERT,), jnp.float32),
            pltpu.VMEM((SLT,), jnp.float32),
            pltpu.VMEM((BROWS, 128), jnp.float32),
            pltpu.VMEM((BROWS, 128), jnp.float32),
            pltpu.VMEM((BROWS,), jnp.int32),
            pltpu.VMEM((BROWS,), jnp.int32),
            pltpu.VMEM((BROWS,), jnp.int32),
            pltpu.VMEM((BROWS,), jnp.int32),
            pltpu.VMEM((BROWS,), jnp.int32),
            pltpu.VMEM((BROWS,), jnp.int32),
            pltpu.VMEM((BROWS,), jnp.int32),
            pltpu.VMEM((BROWS,), jnp.int32),
            pltpu.VMEM((128,), jnp.float32),
            pltpu.SemaphoreType.DMA,
            pltpu.SemaphoreType.DMA,
            pltpu.SemaphoreType.DMA,
            pltpu.SemaphoreType.DMA,
            pltpu.SemaphoreType.DMA,
            pltpu.SemaphoreType.DMA,
            pltpu.SemaphoreType.DMA,
            pltpu.SemaphoreType.DMA,
            pltpu.SemaphoreType.DMA,
            pltpu.SemaphoreType.DMA,
            pltpu.SemaphoreType.DMA,
            pltpu.SemaphoreType.DMA,
            pltpu.VMEM_SHARED((NPAD, 128), jnp.float32),
        ],
    )(xp_flat, src, dst, w_all, bias)

    return out


# speedup vs baseline: 1.0116x; 1.0116x over previous
import jax
import jax.numpy as jnp
from jax import lax
from jax.experimental import pallas as pl
from jax.experimental.pallas import tpu as pltpu
from jax.experimental.pallas import tpu_sc as plsc

N = 10000
E = 160000
D_IN = 256
HEADS = 4
C_OUT = 256
NEG_SLOPE = 0.2

NC, NS, LANES = 2, 16, 16
NPAD = 10240
SLT = NPAD // NS
EPAD = E + NPAD
ERT = E // NS
RCHUNK = 2000
FB = 8
BROWS = 80


def _iota16():
    return lax.iota(jnp.int32, LANES)


def _matmul_body(x_ref, w_ref, o_ref, a_ref):
    o_ref[0] = jnp.dot(x_ref[...], w_ref[...],
                       preferred_element_type=jnp.float32)
    @pl.when(pl.program_id(1) == FB)
    def _():
        a_ref[0] = jnp.dot(x_ref[...], w_ref[:, 0:4],
                           preferred_element_type=jnp.float32)
        a_ref[1] = jnp.dot(x_ref[...], w_ref[:, 4:8],
                           preferred_element_type=jnp.float32)


def _edge_alpha(a_v, sidx, didx, k):
    a_s = plsc.load_gather(a_v, [sidx * 4 + k])
    a_d = plsc.load_gather(a_v, [didx * 4 + (2 + k)])
    al = a_s + a_d
    al = jnp.maximum(al, NEG_SLOPE * al)
    return jnp.exp(al)


def _softmax_body(a_hbm, src_hbm, dst_hbm, w_hbm,
                  a_v, dn_v, src_v, dst_v, wbuf_v, tmp_v, red_v, ea_v,
                  den_sh):
    c = lax.axis_index("c")
    s = lax.axis_index("s")
    DN = NPAD * 2
    SLICE = DN // NS

    pltpu.sync_copy(a_hbm.at[pl.ds(c * (N * 4), N * 4)], a_v)

    def _zero_dn(i, _):
        dn_v[pl.ds(i * LANES, LANES)] = jnp.zeros((LANES,), jnp.float32)
        return 0
    lax.fori_loop(0, DN // LANES, _zero_dn, 0)

    def _chunk1(ci, _):
        e0 = s * ERT + ci * RCHUNK
        pltpu.sync_copy(src_hbm.at[pl.ds(e0, RCHUNK)], src_v)
        pltpu.sync_copy(dst_hbm.at[pl.ds(e0, RCHUNK)], dst_v)

        def _vec(vi, _):
            sidx = src_v[pl.ds(vi * LANES, LANES)]
            didx = dst_v[pl.ds(vi * LANES, LANES)]
            for k in range(2):
                ea = _edge_alpha(a_v, sidx, didx, k)
                ea_v[pl.ds(k * ERT + ci * RCHUNK + vi * LANES, LANES)] = ea
                plsc.addupdate_scatter(dn_v, [didx * 2 + k], ea)
            return 0
        lax.fori_loop(0, RCHUNK // LANES, _vec, 0)
        return 0
    lax.fori_loop(0, ERT // RCHUNK, _chunk1, 0)

    def _slvec(vi, _):
        q = s * SLT + vi * LANES + _iota16()
        valid = q < N
        qc = jnp.where(valid, q, N - 1)
        for k in range(2):
            ea = _edge_alpha(a_v, qc, qc, k)
            ea = jnp.where(valid, ea, 0.0)
            plsc.addupdate_scatter(dn_v, [qc * 2 + k], ea)
        return 0
    lax.fori_loop(0, SLT // LANES, _slvec, 0)

    pltpu.sync_copy(dn_v, den_sh.at[pl.ds(s * DN, DN)])
    plsc.subcore_barrier()
    off = s * SLICE
    pltpu.sync_copy(den_sh.at[pl.ds(off, SLICE)], red_v)

    def _racc(t, _):
        pltpu.sync_copy(den_sh.at[pl.ds(t * DN + off, SLICE)], tmp_v)
        def _vadd(v, _):
            sl = pl.ds(v * LANES, LANES)
            red_v[sl] = red_v[sl] + tmp_v[sl]
            return 0
        lax.fori_loop(0, SLICE // LANES, _vadd, 0)
        return 0
    lax.fori_loop(1, NS, _racc, 0)

    pltpu.sync_copy(red_v, den_sh.at[pl.ds(NS * DN + off, SLICE)])
    plsc.subcore_barrier()
    pltpu.sync_copy(den_sh.at[pl.ds(NS * DN, DN)], dn_v)

    def _chunk2(ci, _):
        e0 = s * ERT + ci * RCHUNK
        pltpu.sync_copy(dst_hbm.at[pl.ds(e0, RCHUNK)], dst_v)

        def _vec(vi, _):
            didx = dst_v[pl.ds(vi * LANES, LANES)]
            for k in range(2):
                ea = ea_v[pl.ds(k * ERT + ci * RCHUNK + vi * LANES, LANES)]
                den = plsc.load_gather(dn_v, [didx * 2 + k])
                wbuf_v[pl.ds(k * RCHUNK + vi * LANES, LANES)] = ea / den
            return 0
        lax.fori_loop(0, RCHUNK // LANES, _vec, 0)
        for k in range(2):
            pltpu.sync_copy(wbuf_v.at[pl.ds(k * RCHUNK, RCHUNK)],
                            w_hbm.at[pl.ds((2 * c + k) * EPAD + e0, RCHUNK)])
        return 0
    lax.fori_loop(0, ERT // RCHUNK, _chunk2, 0)

    def _slvec2(vi, _):
        q = s * SLT + vi * LANES + _iota16()
        valid = q < N
        qc = jnp.where(valid, q, N - 1)
        for k in range(2):
            ea = _edge_alpha(a_v, qc, qc, k)
            den = plsc.load_gather(dn_v, [qc * 2 + k])
            w = jnp.where(valid, ea / den, 0.0)
            wbuf_v[pl.ds(k * RCHUNK + vi * LANES, LANES)] = w
        return 0
    lax.fori_loop(0, SLT // LANES, _slvec2, 0)
    for k in range(2):
        pltpu.sync_copy(
            wbuf_v.at[pl.ds(k * RCHUNK, SLT)],
            w_hbm.at[pl.ds((2 * c + k) * EPAD + E + s * SLT, SLT)])


def _agg_body(xp_hbm, src_hbm, dst_hbm, w_hbm, bias_hbm, out_hbm,
              w_v, wsl_v, rows_a, rows_b,
              gq0a, gq0b, gq1a, gq1b, sq0a, sq0b, sq1a, sq1b,
              bias_v, sem_i0a, sem_i0b, sem_i1a, sem_i1b,
              sem_d0a, sem_d0b, sem_d1a, sem_d1b,
              sem_ga, sem_gb, sem_sa, sem_sb, acc_sh):
    c = lax.axis_index("c")
    s = lax.axis_index("s")
    r0 = s * SLT
    e_base = s * ERT
    NB = ERT // BROWS
    NGI = (NB - 1) // 4

    def _scale_rows(rows, wref, b0):
        def _scale(rh, _):
            r = rh * 2
            wv0 = plsc.load_gather(
                wref, [jnp.full((LANES,), b0, jnp.int32) + r])
            wv1 = plsc.load_gather(
                wref, [jnp.full((LANES,), b0 + 1, jnp.int32) + r])
            for k in range(8):
                sl = pl.ds(k * LANES, LANES)
                rows[r, sl] = rows[r, sl] * wv0
            for k in range(8):
                sl = pl.ds(k * LANES, LANES)
                rows[r + 1, sl] = rows[r + 1, sl] * wv1
            return 0
        lax.fori_loop(0, BROWS // 2, _scale, 0)

    def _add_fbn(gq, fb):
        def _add(v, _):
            sl = pl.ds(v * LANES, LANES)
            gq[sl] = gq[sl] + fb * N
            return 0
        lax.fori_loop(0, BROWS // LANES, _add, 0)

    def _pass(j, _):
        fb = 4 * c + j
        h = fb // 2

        pltpu.sync_copy(w_hbm.at[pl.ds(h * EPAD + e_base, ERT)], w_v)
        pltpu.sync_copy(w_hbm.at[pl.ds(h * EPAD + E + s * SLT, SLT)], wsl_v)
        pltpu.sync_copy(bias_hbm.at[pl.ds(fb * 128, 128)], bias_v)

        def _sl_init(wbase, count_half):
            def _one(rh, _):
                r = rh * 2
                wv0 = plsc.load_gather(
                    wsl_v, [jnp.full((LANES,), wbase, jnp.int32) + r])
                wv1 = plsc.load_gather(
                    wsl_v, [jnp.full((LANES,), wbase + 1, jnp.int32) + r])
                for k in range(8):
                    sl = pl.ds(k * LANES, LANES)
                    rows_a[r, sl] = rows_a[r, sl] * wv0 + bias_v[sl]
                for k in range(8):
                    sl = pl.ds(k * LANES, LANES)
                    rows_a[r + 1, sl] = rows_a[r + 1, sl] * wv1 + bias_v[sl]
                return 0
            lax.fori_loop(0, count_half, _one, 0)

        def _init(zi, _):
            rr = r0 + zi * 64
            @pl.when(rr + 64 <= N)
            def _():
                pltpu.sync_copy(xp_hbm.at[pl.ds(fb * N + rr, 64)],
                                rows_a.at[pl.ds(0, 64)])
                _sl_init(zi * 64, 32)
                pltpu.sync_copy(rows_a.at[pl.ds(0, 64)],
                                acc_sh.at[pl.ds(rr, 64)])
            @pl.when(rr == (N // 64) * 64)
            def _():
                nrem = N - (N // 64) * 64
                pltpu.sync_copy(xp_hbm.at[pl.ds(fb * N + rr, nrem)],
                                rows_a.at[pl.ds(0, nrem)])
                _sl_init(zi * 64, nrem // 2)
                pltpu.sync_copy(rows_a.at[pl.ds(0, nrem)],
                                acc_sh.at[pl.ds(rr, nrem)])
            return 0
        lax.fori_loop(0, SLT // 64, _init, 0)
        plsc.subcore_barrier()

        def _pre_g(b, gq, si):
            pltpu.async_copy(
                src_hbm.at[pl.ds(e_base + b * BROWS, BROWS)], gq, si)

        def _pre_s(b, sq, sd):
            pltpu.async_copy(
                dst_hbm.at[pl.ds(e_base + b * BROWS, BROWS)], sq, sd)

        _pre_g(0, gq0a, sem_i0a); _pre_s(0, sq0a, sem_d0a)
        _pre_g(1, gq0b, sem_i0b); _pre_s(1, sq0b, sem_d0b)
        _pre_g(2, gq1a, sem_i1a); _pre_s(2, sq1a, sem_d1a)
        _pre_g(3, gq1b, sem_i1b); _pre_s(3, sq1b, sem_d1b)

        def _group(g, _):
            b0 = 4 * g

            @pl.when(g > 0)
            def _():
                pltpu.make_async_copy(rows_a, acc_sh.at[sq1a], sem_sa).wait()
                _pre_s(b0 + 2, sq1a, sem_d1a)
                pltpu.make_async_copy(rows_b, acc_sh.at[sq1b], sem_sb).wait()
                _pre_s(b0 + 3, sq1b, sem_d1b)

            def _wait_g(gq, si, boff):
                e0 = e_base + (b0 + boff) * BROWS
                pltpu.make_async_copy(
                    src_hbm.at[pl.ds(e0, BROWS)], gq, si).wait()

            def _wait_s(sq, sd, boff):
                e0 = e_base + (b0 + boff) * BROWS
                pltpu.make_async_copy(
                    dst_hbm.at[pl.ds(e0, BROWS)], sq, sd).wait()

            more = g + 1 < NGI

            _wait_g(gq0a, sem_i0a, 0)
            _add_fbn(gq0a, fb)
            d_ga = pltpu.async_copy(xp_hbm.at[gq0a], rows_a, sem_ga)
            _wait_g(gq0b, sem_i0b, 1)
            _add_fbn(gq0b, fb)
            d_gb = pltpu.async_copy(xp_hbm.at[gq0b], rows_b, sem_gb)
            d_ga.wait()
            @pl.when(more)
            def _():
                _pre_g(b0 + 4, gq0a, sem_i0a)
            _scale_rows(rows_a, w_v, b0 * BROWS)
            _wait_s(sq0a, sem_d0a, 0)
            d_sa = pltpu.async_copy(rows_a, acc_sh.at[sq0a], sem_sa, add=True)
            d_gb.wait()
            @pl.when(more)
            def _():
                _pre_g(b0 + 5, gq0b, sem_i0b)
            _scale_rows(rows_b, w_v, (b0 + 1) * BROWS)
            _wait_s(sq0b, sem_d0b, 1)
            d_sb = pltpu.async_copy(rows_b, acc_sh.at[sq0b], sem_sb, add=True)

            _wait_g(gq1a, sem_i1a, 2)
            _add_fbn(gq1a, fb)
            d_sa.wait()
            @pl.when(more)
            def _():
                _pre_s(b0 + 4, sq0a, sem_d0a)
            d_gc = pltpu.async_copy(xp_hbm.at[gq1a], rows_a, sem_ga)
            _wait_g(gq1b, sem_i1b, 3)
            _add_fbn(gq1b, fb)
            d_sb.wait()
            @pl.when(more)
            def _():
                _pre_s(b0 + 5, sq0b, sem_d0b)
            d_gd = pltpu.async_copy(xp_hbm.at[gq1b], rows_b, sem_gb)
            d_gc.wait()
            @pl.when(more)
            def _():
                _pre_g(b0 + 6, gq1a, sem_i1a)
            _scale_rows(rows_a, w_v, (b0 + 2) * BROWS)
            _wait_s(sq1a, sem_d1a, 2)
            pltpu.async_copy(rows_a, acc_sh.at[sq1a], sem_sa, add=True)
            d_gd.wait()
            @pl.when(more)
            def _():
                _pre_g(b0 + 7, gq1b, sem_i1b)
            _scale_rows(rows_b, w_v, (b0 + 3) * BROWS)
            _wait_s(sq1b, sem_d1b, 3)
            pltpu.async_copy(rows_b, acc_sh.at[sq1b], sem_sb, add=True)
            return 0
        lax.fori_loop(0, NGI, _group, 0)

        pltpu.make_async_copy(rows_a, acc_sh.at[sq1a], sem_sa).wait()
        pltpu.make_async_copy(rows_b, acc_sh.at[sq1b], sem_sb).wait()

        bT = NB - 1
        eT = e_base + bT * BROWS
        pltpu.sync_copy(src_hbm.at[pl.ds(eT, BROWS)], gq0a)
        pltpu.sync_copy(dst_hbm.at[pl.ds(eT, BROWS)], sq0a)
        _add_fbn(gq0a, fb)
        pltpu.sync_copy(xp_hbm.at[gq0a], rows_a)
        _scale_rows(rows_a, w_v, bT * BROWS)
        pltpu.sync_copy(rows_a, acc_sh.at[sq0a], add=True)

        plsc.subcore_barrier()

        def _wchunk(zi, _):
            rr = r0 + zi * 64
            @pl.when(rr + 64 <= N)
            def _():
                pltpu.sync_copy(acc_sh.at[pl.ds(rr, 64)],
                                out_hbm.at[pl.ds(rr, 64),
                                           pl.ds(fb * 128, 128)])
            @pl.when(rr == (N // 64) * 64)
            def _():
                nrem = N - (N // 64) * 64
                pltpu.sync_copy(acc_sh.at[pl.ds(rr, nrem)],
                                out_hbm.at[pl.ds(rr, nrem),
                                           pl.ds(fb * 128, 128)])
            return 0
        lax.fori_loop(0, SLT // 64, _wchunk, 0)
        plsc.subcore_barrier()
        return 0
    lax.fori_loop(0, 4, _pass, 0)


@jax.jit
def kernel(x, edge_index, W, att_src, att_dst, bias):
    ei = edge_index.astype(jnp.int32)
    src = ei[0]
    dst = ei[1]

    w3 = W.reshape(D_IN, HEADS, C_OUT)
    weff_src = jnp.einsum('dhc,hc->dh', w3, att_src)
    weff_dst = jnp.einsum('dhc,hc->dh', w3, att_dst)
    wsel = jnp.concatenate(
        [weff_src[:, 0:2], weff_dst[:, 0:2],
         weff_src[:, 2:4], weff_dst[:, 2:4]], axis=1)
    wcat = jnp.concatenate(
        [W, wsel, jnp.zeros((D_IN, 120), jnp.float32)], axis=1)

    RT = 1000
    xp_all, a_sc = pl.pallas_call(
        _matmul_body,
        grid=(N // RT, FB + 1),
        in_specs=[
            pl.BlockSpec((RT, D_IN), lambda i, j: (i, 0)),
            pl.BlockSpec((D_IN, 128), lambda i, j: (0, j)),
        ],
        out_specs=[
            pl.BlockSpec((1, RT, 128), lambda i, j: (j, i, 0)),
            pl.BlockSpec((2, RT, 4), lambda i, j: (0, i, 0)),
        ],
        out_shape=[
            jax.ShapeDtypeStruct((FB + 1, N, 128), jnp.float32),
            jax.ShapeDtypeStruct((2, N, 4), jnp.float32),
        ],
    )(x, wcat)

    a_all = a_sc.reshape(2 * N * 4)
    xp_flat = xp_all.reshape((FB + 1) * N, 128)

    mesh = plsc.VectorSubcoreMesh(core_axis_name="c", subcore_axis_name="s",
                                  num_cores=NC, num_subcores=NS)

    w_all = pl.kernel(
        _softmax_body,
        out_type=jax.ShapeDtypeStruct((HEADS * EPAD,), jnp.float32),
        mesh=mesh,
        compiler_params=pltpu.CompilerParams(needs_layout_passes=False),
        scratch_types=[
            pltpu.VMEM((N * 4,), jnp.float32),
            pltpu.VMEM((NPAD * 2,), jnp.float32),
            pltpu.VMEM((RCHUNK,), jnp.int32),
            pltpu.VMEM((RCHUNK,), jnp.int32),
            pltpu.VMEM((2 * RCHUNK,), jnp.float32),
            pltpu.VMEM((NPAD * 2 // NS,), jnp.float32),
            pltpu.VMEM((NPAD * 2 // NS,), jnp.float32),
            pltpu.VMEM((2 * ERT,), jnp.float32),
            pltpu.VMEM_SHARED(((NS + 1) * NPAD * 2,), jnp.float32),
        ],
    )(a_all, src, dst)

    out = pl.kernel(
        _agg_body,
        out_type=jax.ShapeDtypeStruct((N, HEADS * C_OUT), jnp.float32),
        mesh=mesh,
        compiler_params=pltpu.CompilerParams(needs_layout_passes=False),
        scratch_types=[
            pltpu.VMEM((ERT,), jnp.float32),
            pltpu.VMEM((SLT,), jnp.float32),
            pltpu.VMEM((BROWS, 128), jnp.float32),
            pltpu.VMEM((BROWS, 128), jnp.float32),
            pltpu.VMEM((BROWS,), jnp.int32),
            pltpu.VMEM((BROWS,), jnp.int32),
            pltpu.VMEM((BROWS,), jnp.int32),
            pltpu.VMEM((BROWS,), jnp.int32),
            pltpu.VMEM((BROWS,), jnp.int32),
            pltpu.VMEM((BROWS,), jnp.int32),
            pltpu.VMEM((BROWS,), jnp.int32),
            pltpu.VMEM((BROWS,), jnp.int32),
            pltpu.VMEM((128,), jnp.float32),
            pltpu.SemaphoreType.DMA,
            pltpu.SemaphoreType.DMA,
            pltpu.SemaphoreType.DMA,
            pltpu.SemaphoreType.DMA,
            pltpu.SemaphoreType.DMA,
            pltpu.SemaphoreType.DMA,
            pltpu.SemaphoreType.DMA,
            pltpu.SemaphoreType.DMA,
            pltpu.SemaphoreType.DMA,
            pltpu.SemaphoreType.DMA,
            pltpu.SemaphoreType.DMA,
            pltpu.SemaphoreType.DMA,
            pltpu.VMEM_SHARED((NPAD, 128), jnp.float32),
        ],
    )(xp_flat, src, dst, w_all, bias)

    return out
